# baseline (speedup 1.0000x reference)
import functools

import jax
import jax.numpy as jnp
from jax import lax
from jax.experimental import pallas as pl
from jax.experimental.pallas import tpu as pltpu
from jax.experimental.pallas import tpu_sc as plsc

BATCH = 16384
EMBED = 16
NW = 32
BPW = BATCH // NW
NGRP = BPW // 16


def _gather_body(su_hbm, pu_hbm, si_hbm, pi_hbm, uttab_hbm, ittab_hbm,
                 uout_hbm, iout_hbm,
                 uidx_v, upos_v, iidx_v, ipos_v, slab_v, uflat_v, iflat_v,
                 *sems):
    wid = lax.axis_index("s") * 2 + lax.axis_index("c")
    base = wid * BPW
    pltpu.sync_copy(su_hbm.at[pl.ds(base, BPW)], uidx_v)
    pltpu.sync_copy(pu_hbm.at[pl.ds(base, BPW)], upos_v)
    pltpu.sync_copy(si_hbm.at[pl.ds(base, BPW)], iidx_v)
    pltpu.sync_copy(pi_hbm.at[pl.ds(base, BPW)], ipos_v)
    lanes = lax.iota(jnp.int32, 16)
    wsem = sems[16]

    for t, (idx_v, pos_v, tab_hbm, flat_v, out_hbm) in enumerate(
        ((uidx_v, upos_v, uttab_hbm, uflat_v, uout_hbm),
         (iidx_v, ipos_v, ittab_hbm, iflat_v, iout_hbm))):

        def grp(g, carry, idx_v=idx_v, pos_v=pos_v, tab_hbm=tab_hbm,
                flat_v=flat_v, out_hbm=out_hbm):
            start = g * 16
            iv = idx_v[pl.ds(start, 16)]
            col = iv & 127
            off = (iv >> 7) * 128
            pv = idx_v[pl.ds(jnp.maximum(start - 1, 0), 16)]
            headi = jnp.where(
                ((iv >> 7) != (pv >> 7)) | (lanes == 0) | (g == 0), 1, 0
            ).astype(jnp.int32)
            slotv = plsc.cummax(jnp.where(headi == 1, lanes, 0))
            for k in range(16):
                @pl.when(headi[k] == 1)
                def _fire(k=k, off=off, tab_hbm=tab_hbm):
                    pltpu.async_copy(
                        tab_hbm.at[:, pl.ds(pl.multiple_of(off[k], 128), 128)],
                        slab_v.at[t * 16 + k], sems[k])
            for k in range(16):
                @pl.when(headi[k] == 1)
                def _drain(k=k, tab_hbm=tab_hbm):
                    pltpu.make_async_copy(
                        tab_hbm.at[:, pl.ds(0, 128)],
                        slab_v.at[t * 16 + k], sems[k]).wait()
                v = plsc.load_gather(
                    slab_v,
                    [jnp.full((16,), t * 16 + slotv[k], jnp.int32), lanes,
                     jnp.full((16,), col[k], jnp.int32)])
                plsc.store_scatter(flat_v, [(start + k) * EMBED + lanes], v)
            posv = pos_v[pl.ds(start, 16)]
            for k in range(16):
                pltpu.async_copy(
                    flat_v.at[pl.ds((start + k) * EMBED, EMBED)],
                    out_hbm.at[pl.ds(posv[k] * EMBED, EMBED)], wsem)
            return carry

        lax.fori_loop(0, NGRP, grp, 0)
        pltpu.make_async_copy(
            flat_v, out_hbm.at[pl.ds(0, BPW * EMBED)], wsem).wait()


@functools.cache
def _gather():
    return pl.kernel(
        _gather_body,
        mesh=plsc.VectorSubcoreMesh(core_axis_name="c", subcore_axis_name="s"),
        compiler_params=pltpu.CompilerParams(needs_layout_passes=False),
        out_type=[
            jax.ShapeDtypeStruct((BATCH * EMBED,), jnp.float32),
            jax.ShapeDtypeStruct((BATCH * EMBED,), jnp.float32),
        ],
        scratch_types=(
            [pltpu.VMEM((BPW,), jnp.int32),
             pltpu.VMEM((BPW,), jnp.int32),
             pltpu.VMEM((BPW,), jnp.int32),
             pltpu.VMEM((BPW,), jnp.int32),
             pltpu.VMEM((32, EMBED, 128), jnp.float32),
             pltpu.VMEM((BPW * EMBED,), jnp.float32),
             pltpu.VMEM((BPW * EMBED,), jnp.float32)]
            + [pltpu.SemaphoreType.DMA] * 17
        ),
    )


B_BLK = 16384
R_BLK = B_BLK // 8


def _mlp_body(xu_ref, xi_ref, m1u_ref, m1i_ref, b1_ref, m2_ref, b2_ref,
              m3_ref, b3_ref, out_ref):
    h = (jnp.dot(xu_ref[...], m1u_ref[...], preferred_element_type=jnp.float32)
         + jnp.dot(xi_ref[...], m1i_ref[...], preferred_element_type=jnp.float32)
         + b1_ref[...])
    h = jnp.maximum(h, 0.0)
    h = jnp.dot(h, m2_ref[...], preferred_element_type=jnp.float32) + b2_ref[...]
    h = jnp.maximum(h, 0.0)
    out_ref[...] = (jnp.dot(h, m3_ref[...], preferred_element_type=jnp.float32)
                    + b3_ref[...])


def _mlp(xu, xi, m1u, m1i, b1t, m2, b2t, m3, b3t):
    grid = (BATCH // B_BLK,)
    full = lambda shape: pl.BlockSpec(shape, lambda i: (0, 0))
    return pl.pallas_call(
        _mlp_body,
        grid=grid,
        in_specs=[
            pl.BlockSpec((R_BLK, 128), lambda i: (i, 0)),
            pl.BlockSpec((R_BLK, 128), lambda i: (i, 0)),
            full((128, 128)),
            full((128, 128)),
            full((1, 128)),
            full((128, 64)),
            full((1, 64)),
            full((64, 8)),
            full((1, 8)),
        ],
        out_specs=pl.BlockSpec((R_BLK, 8), lambda i: (i, 0)),
        out_shape=jax.ShapeDtypeStruct((BATCH // 8, 8), jnp.float32),
    )(xu, xi, m1u, m1i, b1t, m2, b2t, m3, b3t)


def kernel(user, item, user_table, item_table, W1, b1, W2, b2, W3, b3):
    user32 = user.astype(jnp.int32)
    item32 = item.astype(jnp.int32)
    keys = jnp.stack([user32, item32])
    pos2 = jnp.broadcast_to(lax.iota(jnp.int32, BATCH), (2, BATCH))
    sk, sp = lax.sort((keys, pos2), dimension=1, num_keys=1)
    su, si = sk[0], sk[1]
    pu, pi_ = sp[0], sp[1]
    u_flat, i_flat = _gather()(su, pu, si, pi_, user_table.T, item_table.T)
    xu = u_flat.reshape(BATCH // 8, 128)
    xi = i_flat.reshape(BATCH // 8, 128)
    eye8 = jnp.eye(8, dtype=jnp.float32)
    m1u = jnp.kron(eye8, W1[:EMBED])
    m1i = jnp.kron(eye8, W1[EMBED:])
    b1t = jnp.tile(b1, 8).reshape(1, 128)
    m2 = jnp.kron(eye8, W2)
    b2t = jnp.tile(b2, 8).reshape(1, 64)
    m3 = jnp.kron(eye8, W3)
    b3t = jnp.tile(b3, 8).reshape(1, 8)
    out = _mlp(xu, xi, m1u, m1i, b1t, m2, b2t, m3, b3t)
    return out.reshape(BATCH)


# speedup vs baseline: 1.3245x; 1.3245x over previous
import functools

import jax
import jax.numpy as jnp
from jax import lax
from jax.experimental import pallas as pl
from jax.experimental.pallas import tpu as pltpu
from jax.experimental.pallas import tpu_sc as plsc

BATCH = 16384
EMBED = 16
NW = 32
BPW = BATCH // NW
NGRP = BPW // 16


def _gather_body(su_hbm, pu_hbm, si_hbm, pi_hbm, uttab_hbm, ittab_hbm,
                 uout_hbm, iout_hbm,
                 uidx_v, upos_v, iidx_v, ipos_v, slab_v, uflat_v, iflat_v,
                 *sems):
    wid = lax.axis_index("s") * 2 + lax.axis_index("c")
    base = wid * BPW
    pltpu.sync_copy(su_hbm.at[pl.ds(base, BPW)], uidx_v)
    pltpu.sync_copy(pu_hbm.at[pl.ds(base, BPW)], upos_v)
    pltpu.sync_copy(si_hbm.at[pl.ds(base, BPW)], iidx_v)
    pltpu.sync_copy(pi_hbm.at[pl.ds(base, BPW)], ipos_v)
    lanes = lax.iota(jnp.int32, 16)
    wsem = sems[16]

    for t, (idx_v, pos_v, tab_hbm, flat_v, out_hbm) in enumerate(
        ((uidx_v, upos_v, uttab_hbm, uflat_v, uout_hbm),
         (iidx_v, ipos_v, ittab_hbm, iflat_v, iout_hbm))):

        def grp(g, carry, idx_v=idx_v, pos_v=pos_v, tab_hbm=tab_hbm,
                flat_v=flat_v, out_hbm=out_hbm):
            start = g * 16
            iv = idx_v[pl.ds(start, 16)]
            col = iv & 127
            off = (iv >> 7) * 128
            pv = idx_v[pl.ds(jnp.maximum(start - 1, 0), 16)]
            headi = jnp.where(
                ((iv >> 7) != (pv >> 7)) | (lanes == 0) | (g == 0), 1, 0
            ).astype(jnp.int32)
            slotv = plsc.cummax(jnp.where(headi == 1, lanes, 0))
            for k in range(16):
                @pl.when(headi[k] == 1)
                def _fire(k=k, off=off, tab_hbm=tab_hbm):
                    pltpu.async_copy(
                        tab_hbm.at[:, pl.ds(pl.multiple_of(off[k], 128), 128)],
                        slab_v.at[t * 16 + k], sems[k])
            for k in range(16):
                @pl.when(headi[k] == 1)
                def _drain(k=k, tab_hbm=tab_hbm):
                    pltpu.make_async_copy(
                        tab_hbm.at[:, pl.ds(0, 128)],
                        slab_v.at[t * 16 + k], sems[k]).wait()
                v = plsc.load_gather(
                    slab_v,
                    [jnp.full((16,), t * 16 + slotv[k], jnp.int32), lanes,
                     jnp.full((16,), col[k], jnp.int32)])
                plsc.store_scatter(flat_v, [(start + k) * EMBED + lanes], v)
            posv = pos_v[pl.ds(start, 16)]
            for k in range(16):
                pltpu.async_copy(
                    flat_v.at[pl.ds((start + k) * EMBED, EMBED)],
                    out_hbm.at[pl.ds(posv[k] * EMBED, EMBED)], wsem)
            return carry

        lax.fori_loop(0, NGRP, grp, 0)
        pltpu.make_async_copy(
            flat_v, out_hbm.at[pl.ds(0, BPW * EMBED)], wsem).wait()


@functools.cache
def _gather():
    return pl.kernel(
        _gather_body,
        mesh=plsc.VectorSubcoreMesh(core_axis_name="c", subcore_axis_name="s"),
        compiler_params=pltpu.CompilerParams(needs_layout_passes=False),
        out_type=[
            jax.ShapeDtypeStruct((BATCH * EMBED,), jnp.float32),
            jax.ShapeDtypeStruct((BATCH * EMBED,), jnp.float32),
        ],
        scratch_types=(
            [pltpu.VMEM((BPW,), jnp.int32),
             pltpu.VMEM((BPW,), jnp.int32),
             pltpu.VMEM((BPW,), jnp.int32),
             pltpu.VMEM((BPW,), jnp.int32),
             pltpu.VMEM((32, EMBED, 128), jnp.float32),
             pltpu.VMEM((BPW * EMBED,), jnp.float32),
             pltpu.VMEM((BPW * EMBED,), jnp.float32)]
            + [pltpu.SemaphoreType.DMA] * 17
        ),
    )


B_BLK = 16384
R_BLK = B_BLK // 8


def _mlp_body(xu_ref, xi_ref, m1u_ref, m1i_ref, b1_ref, m2_ref, b2_ref,
              m3_ref, b3_ref, out_ref):
    h = (jnp.dot(xu_ref[...], m1u_ref[...], preferred_element_type=jnp.float32)
         + jnp.dot(xi_ref[...], m1i_ref[...], preferred_element_type=jnp.float32)
         + b1_ref[...])
    h = jnp.maximum(h, 0.0)
    h = jnp.dot(h, m2_ref[...], preferred_element_type=jnp.float32) + b2_ref[...]
    h = jnp.maximum(h, 0.0)
    out_ref[...] = (jnp.dot(h, m3_ref[...], preferred_element_type=jnp.float32)
                    + b3_ref[...])


def _mlp(xu, xi, m1u, m1i, b1t, m2, b2t, m3, b3t):
    grid = (BATCH // B_BLK,)
    full = lambda shape: pl.BlockSpec(shape, lambda i: (0, 0))
    return pl.pallas_call(
        _mlp_body,
        grid=grid,
        in_specs=[
            pl.BlockSpec((R_BLK, 128), lambda i: (i, 0)),
            pl.BlockSpec((R_BLK, 128), lambda i: (i, 0)),
            full((128, 128)),
            full((128, 128)),
            full((1, 128)),
            full((128, 64)),
            full((1, 64)),
            full((64, 8)),
            full((1, 8)),
        ],
        out_specs=pl.BlockSpec((R_BLK, 8), lambda i: (i, 0)),
        out_shape=jax.ShapeDtypeStruct((BATCH // 8, 8), jnp.float32),
    )(xu, xi, m1u, m1i, b1t, m2, b2t, m3, b3t)


def kernel(user, item, user_table, item_table, W1, b1, W2, b2, W3, b3):
    user32 = user.astype(jnp.int32)
    item32 = item.astype(jnp.int32)
    pos = lax.iota(jnp.int32, BATCH)
    su, pu = lax.sort((user32, pos), num_keys=1)
    si, pi_ = lax.sort((item32, pos), num_keys=1)
    u_flat, i_flat = _gather()(su, pu, si, pi_, user_table.T, item_table.T)
    xu = u_flat.reshape(BATCH // 8, 128)
    xi = i_flat.reshape(BATCH // 8, 128)
    eye8 = jnp.eye(8, dtype=jnp.float32)
    m1u = jnp.kron(eye8, W1[:EMBED])
    m1i = jnp.kron(eye8, W1[EMBED:])
    b1t = jnp.tile(b1, 8).reshape(1, 128)
    m2 = jnp.kron(eye8, W2)
    b2t = jnp.tile(b2, 8).reshape(1, 64)
    m3 = jnp.kron(eye8, W3)
    b3t = jnp.tile(b3, 8).reshape(1, 8)
    out = _mlp(xu, xi, m1u, m1i, b1t, m2, b2t, m3, b3t)
    return out.reshape(BATCH)
